# baseline (speedup 1.0000x reference)
import functools

import jax
import jax.numpy as jnp
from jax import lax
from jax.experimental import pallas as pl
from jax.experimental.pallas import tpu as pltpu
from jax.experimental.pallas import tpu_sc as plsc

B = 16384
D = 32
NC = 2
NS = 16
NW = NC * NS
BPW = B // NW


def _pmf_body(uidx_hbm, iidx_hbm, uemb_hbm, iemb_hbm, out_hbm,
              uidx_v, iidx_v, urows_v, irows_v, out_v, sem):
    c = lax.axis_index("c")
    s = lax.axis_index("s")
    wid = s * NC + c
    base = wid * BPW

    pltpu.sync_copy(uidx_hbm.at[pl.ds(base, BPW)], uidx_v)
    pltpu.sync_copy(iidx_hbm.at[pl.ds(base, BPW)], iidx_v)

    cu = pltpu.async_copy(uemb_hbm.at[uidx_v], urows_v, sem)
    ci = pltpu.async_copy(iemb_hbm.at[iidx_v], irows_v, sem)
    cu.wait()
    ci.wait()

    lane = lax.iota(jnp.int32, 16)

    def merge(a, b, k):
        perm = lane ^ k
        mask = (lane & k) == 0
        a_s = a.at[perm].get(mode="promise_in_bounds")
        b_s = b.at[perm].get(mode="promise_in_bounds")
        return jnp.where(mask, a, b_s) + jnp.where(mask, a_s, b)

    def group(g, carry):
        base_r = g * 16
        vs = []
        for r in range(16):
            u0 = urows_v[base_r + r, pl.ds(0, 16)]
            i0 = irows_v[base_r + r, pl.ds(0, 16)]
            u1 = urows_v[base_r + r, pl.ds(16, 16)]
            i1 = irows_v[base_r + r, pl.ds(16, 16)]
            vs.append(u0 * i0 + u1 * i1)
        for k in (1, 2, 4, 8):
            vs = [merge(vs[2 * j], vs[2 * j + 1], k)
                  for j in range(len(vs) // 2)]
        out_v[pl.ds(base_r, 16)] = vs[0]
        return carry

    lax.fori_loop(0, BPW // 16, group, 0)

    pltpu.sync_copy(out_v, out_hbm.at[pl.ds(base, BPW)])


@functools.partial(jax.jit, donate_argnums=())
def _pmf(users_index, items_index, user_emb, item_emb):
    mesh = plsc.VectorSubcoreMesh(core_axis_name="c", subcore_axis_name="s")
    f = functools.partial(
        pl.kernel,
        mesh=mesh,
        out_type=jax.ShapeDtypeStruct((B,), jnp.float32),
        compiler_params=pltpu.CompilerParams(use_tc_tiling_on_sc=False),
        scratch_types=[
            pltpu.VMEM((BPW,), jnp.int32),
            pltpu.VMEM((BPW,), jnp.int32),
            pltpu.VMEM((BPW, D), jnp.float32),
            pltpu.VMEM((BPW, D), jnp.float32),
            pltpu.VMEM((BPW,), jnp.float32),
            pltpu.SemaphoreType.DMA,
        ],
    )(_pmf_body)
    return f(users_index, items_index, user_emb, item_emb)


def kernel(users_index, items_index, user_emb, item_emb):
    return _pmf(users_index, items_index, user_emb, item_emb)
